# Initial kernel scaffold; baseline (speedup 1.0000x reference)
import jax
import jax.numpy as jnp
from jax.experimental import pallas as pl


def kernel(in_feat, edge_index, W1, b1, W2, b2, W3, b3):
    raise NotImplementedError("write your pallas kernel here")


# speedup vs baseline: 6.6917x; 6.6917x over previous
import functools

import jax
import jax.numpy as jnp
from jax import lax
from jax.experimental import pallas as pl
from jax.experimental.pallas import tpu as pltpu
from jax.experimental.pallas import tpu_sc as plsc

N = 10000
E = 320000
D = 128
DO = 64

NC = 2
NS = 16
NW = NC * NS
EPT = E // NW
CH = 80
NCH = EPT // CH
NPAD = 10240
ZR = NPAD // NS
OR = N // NS

_mesh = plsc.VectorSubcoreMesh(core_axis_name="c", subcore_axis_name="s")

_f32 = jnp.float32
_i32 = jnp.int32


@functools.partial(
    pl.kernel,
    out_type=[jax.ShapeDtypeStruct((NC, NPAD), _f32),
              jax.ShapeDtypeStruct((NC, NPAD), _f32)],
    mesh=_mesh,
    compiler_params=pltpu.CompilerParams(needs_layout_passes=False),
    scratch_types=[
        pltpu.VMEM((NCH, CH), _i32),
        pltpu.VMEM((NCH, CH), _i32),
        pltpu.VMEM((NPAD,), _f32),
        pltpu.VMEM((NPAD,), _f32),
        pltpu.VMEM((ZR,), _f32),
        pltpu.VMEM((ZR,), _f32),
        pltpu.VMEM((ZR,), _f32),
        pltpu.VMEM((ZR,), _f32),
        pltpu.VMEM_SHARED((NS, 2, NPAD), _f32),
    ],
)
def _sc_degrees(src_hbm, dst_hbm, dego_out, degi_out,
                src_v, dst_v, dego_v, degi_v,
                acco_v, acci_v, tmpo_v, tmpi_v, slots):
    c = lax.axis_index("c")
    s = lax.axis_index("s")
    wid = s * NC + c
    pltpu.sync_copy(src_hbm.at[wid], src_v)
    pltpu.sync_copy(dst_hbm.at[wid], dst_v)
    z16 = jnp.zeros((16,), _f32)

    def zloop(i, _):
        dego_v[pl.ds(i * 16, 16)] = z16
        degi_v[pl.ds(i * 16, 16)] = z16
        return 0
    lax.fori_loop(0, NPAD // 16, zloop, 0)

    ones16 = jnp.ones((16,), _f32)

    def eloop(i, _):
        for j in range(CH // 16):
            sv = src_v[i, pl.ds(j * 16, 16)]
            dv = dst_v[i, pl.ds(j * 16, 16)]
            plsc.addupdate_scatter(dego_v, [sv], ones16)
            plsc.addupdate_scatter(degi_v, [dv], ones16)
        return 0
    lax.fori_loop(0, NCH, eloop, 0)

    pltpu.sync_copy(dego_v, slots.at[s, 0])
    pltpu.sync_copy(degi_v, slots.at[s, 1])
    plsc.subcore_barrier()

    pltpu.sync_copy(slots.at[0, 0, pl.ds(s * ZR, ZR)], acco_v)
    pltpu.sync_copy(slots.at[0, 1, pl.ds(s * ZR, ZR)], acci_v)

    def red(p, _):
        pltpu.sync_copy(slots.at[p, 0, pl.ds(s * ZR, ZR)], tmpo_v)
        pltpu.sync_copy(slots.at[p, 1, pl.ds(s * ZR, ZR)], tmpi_v)
        for k in range(ZR // 16):
            dk = pl.ds(k * 16, 16)
            acco_v[dk] = acco_v[dk] + tmpo_v[dk]
            acci_v[dk] = acci_v[dk] + tmpi_v[dk]
        return 0
    lax.fori_loop(1, NS, red, 0)

    pltpu.sync_copy(acco_v, dego_out.at[c, pl.ds(s * ZR, ZR)])
    pltpu.sync_copy(acci_v, degi_out.at[c, pl.ds(s * ZR, ZR)])


@functools.partial(
    pl.kernel,
    out_type=[jax.ShapeDtypeStruct((NC, NPAD, D), _f32),
              jax.ShapeDtypeStruct((NC, NPAD), _f32)],
    mesh=_mesh,
    compiler_params=pltpu.CompilerParams(needs_layout_passes=False),
    scratch_types=[
        pltpu.VMEM((CH,), _i32),
        pltpu.VMEM((CH,), _i32),
        pltpu.VMEM((CH, D), _f32),
        pltpu.VMEM((16, D), _f32),
        pltpu.VMEM((N,), _f32),
        pltpu.VMEM((NPAD,), _f32),
        pltpu.VMEM((ZR,), _f32),
        pltpu.VMEM((ZR,), _f32),
        pltpu.VMEM_SHARED((NPAD, D), _f32),
        pltpu.VMEM_SHARED((NS, NPAD), _f32),
        pltpu.SemaphoreType.DMA,
    ],
)
def _sc_agg_c0(h_hbm, src_hbm, dst_hbm, nd_hbm, agg_out, c0_out,
               src_c, dst_c, rows_v, zrow_v, nd_v, c_v,
               cacc_v, ctmp_v, sh_acc, cslots, sem):
    c = lax.axis_index("c")
    s = lax.axis_index("s")
    wid = s * NC + c
    pltpu.sync_copy(nd_hbm, nd_v)

    z16 = jnp.zeros((16,), _f32)
    for i in range(16):
        for j in range(D // 16):
            zrow_v[i, pl.ds(j * 16, 16)] = z16

    def zc(i, _):
        c_v[pl.ds(i * 16, 16)] = z16
        return 0
    lax.fori_loop(0, NPAD // 16, zc, 0)

    def zs(k, _):
        pltpu.sync_copy(zrow_v, sh_acc.at[pl.ds(s * ZR + k * 16, 16)])
        return 0
    lax.fori_loop(0, ZR // 16, zs, 0)
    plsc.subcore_barrier()

    def chunk(j, _):
        pltpu.sync_copy(src_hbm.at[wid, j], src_c)
        pltpu.sync_copy(dst_hbm.at[wid, j], dst_c)
        pltpu.async_copy(h_hbm.at[src_c], rows_v, sem).wait()
        pltpu.sync_copy(rows_v, sh_acc.at[dst_c], add=True)
        for jj in range(CH // 16):
            sv = src_c[pl.ds(jj * 16, 16)]
            dv = dst_c[pl.ds(jj * 16, 16)]
            vals = plsc.load_gather(nd_v, [dv])
            plsc.addupdate_scatter(c_v, [sv], vals)
        return 0
    lax.fori_loop(0, NCH, chunk, 0)

    pltpu.sync_copy(c_v, cslots.at[s])
    plsc.subcore_barrier()

    pltpu.sync_copy(sh_acc.at[pl.ds(s * ZR, ZR)],
                    agg_out.at[c, pl.ds(s * ZR, ZR)])

    pltpu.sync_copy(cslots.at[0, pl.ds(s * ZR, ZR)], cacc_v)

    def cred(p, _):
        pltpu.sync_copy(cslots.at[p, pl.ds(s * ZR, ZR)], ctmp_v)
        for k in range(ZR // 16):
            dk = pl.ds(k * 16, 16)
            cacc_v[dk] = cacc_v[dk] + ctmp_v[dk]
        return 0
    lax.fori_loop(1, NS, cred, 0)

    pltpu.sync_copy(cacc_v, c0_out.at[c, pl.ds(s * ZR, ZR)])


@functools.partial(
    pl.kernel,
    out_type=jax.ShapeDtypeStruct((NC, NPAD, D), _f32),
    mesh=_mesh,
    compiler_params=pltpu.CompilerParams(needs_layout_passes=False),
    scratch_types=[
        pltpu.VMEM((CH,), _i32),
        pltpu.VMEM((CH,), _i32),
        pltpu.VMEM((CH, D), _f32),
        pltpu.VMEM((16, D), _f32),
        pltpu.VMEM_SHARED((NPAD, D), _f32),
        pltpu.SemaphoreType.DMA,
    ],
)
def _sc_agg(h_hbm, src_hbm, dst_hbm, agg_out,
            src_c, dst_c, rows_v, zrow_v, sh_acc, sem):
    c = lax.axis_index("c")
    s = lax.axis_index("s")
    wid = s * NC + c

    z16 = jnp.zeros((16,), _f32)
    for i in range(16):
        for j in range(D // 16):
            zrow_v[i, pl.ds(j * 16, 16)] = z16

    def zs(k, _):
        pltpu.sync_copy(zrow_v, sh_acc.at[pl.ds(s * ZR + k * 16, 16)])
        return 0
    lax.fori_loop(0, ZR // 16, zs, 0)
    plsc.subcore_barrier()

    def chunk(j, _):
        pltpu.sync_copy(src_hbm.at[wid, j], src_c)
        pltpu.sync_copy(dst_hbm.at[wid, j], dst_c)
        pltpu.async_copy(h_hbm.at[src_c], rows_v, sem).wait()
        pltpu.sync_copy(rows_v, sh_acc.at[dst_c], add=True)
        return 0
    lax.fori_loop(0, NCH, chunk, 0)
    plsc.subcore_barrier()

    pltpu.sync_copy(sh_acc.at[pl.ds(s * ZR, ZR)],
                    agg_out.at[c, pl.ds(s * ZR, ZR)])


_BR = 400
_GRID = N // _BR


def _tc_norm_mm_body(dego_a, dego_b, degi_a, degi_b, x, w, ho, nso, ndo):
    ns = lax.rsqrt(jnp.maximum(dego_a[...] + dego_b[...], 1.0))
    nd = lax.rsqrt(jnp.maximum(degi_a[...] + degi_b[...], 1.0))
    ho[...] = jnp.dot(x[...] * ns, w[...],
                      preferred_element_type=_f32,
                      precision=lax.Precision.HIGHEST)
    nso[...] = ns
    ndo[...] = nd


_col1 = pl.BlockSpec((_BR, 1), lambda i: (i, 0))
_rowb = pl.BlockSpec((_BR, D), lambda i: (i, 0))
_wfull = pl.BlockSpec((D, D), lambda i: (0, 0))

_tc_norm_mm = pl.pallas_call(
    _tc_norm_mm_body,
    grid=(_GRID,),
    in_specs=[_col1, _col1, _col1, _col1, _rowb, _wfull],
    out_specs=[_rowb, _col1, _col1],
    out_shape=[jax.ShapeDtypeStruct((N, D), _f32),
               jax.ShapeDtypeStruct((N, 1), _f32),
               jax.ShapeDtypeStruct((N, 1), _f32)],
)


def _tc_mid_body(agg_a, agg_b, nd, ns, b, w, ho):
    h = jnp.maximum((agg_a[...] + agg_b[...]) * nd[...] + b[...], 0.0)
    ho[...] = jnp.dot(h * ns[...], w[...],
                      preferred_element_type=_f32,
                      precision=lax.Precision.HIGHEST)


_bfull = pl.BlockSpec((1, D), lambda i: (0, 0))

_tc_mid = pl.pallas_call(
    _tc_mid_body,
    grid=(_GRID,),
    in_specs=[_rowb, _rowb, _col1, _col1, _bfull, _wfull],
    out_specs=_rowb,
    out_shape=jax.ShapeDtypeStruct((N, D), _f32),
)


def _tc_final_body(agg_a, agg_b, nd, b2, c0a, c0b, ns, w3, b3, out, racc):
    i = pl.program_id(0)

    @pl.when(i == 0)
    def _():
        racc[...] = jnp.zeros_like(racc)

    h2 = jnp.maximum((agg_a[...] + agg_b[...]) * nd[...] + b2[...], 0.0)
    wvec = (c0a[...] + c0b[...]) * ns[...]
    racc[...] += jnp.sum(h2 * wvec, axis=0, keepdims=True)

    @pl.when(i == pl.num_programs(0) - 1)
    def _():
        out[...] = jnp.dot(racc[...] * (1.0 / N), w3[...],
                           preferred_element_type=_f32,
                           precision=lax.Precision.HIGHEST) + b3[...]


_tc_final = pl.pallas_call(
    _tc_final_body,
    grid=(_GRID,),
    in_specs=[_rowb, _rowb, _col1, _bfull, _col1, _col1, _col1,
              pl.BlockSpec((D, DO), lambda i: (0, 0)),
              pl.BlockSpec((1, DO), lambda i: (0, 0))],
    out_specs=pl.BlockSpec((1, DO), lambda i: (0, 0)),
    out_shape=jax.ShapeDtypeStruct((1, DO), _f32),
    scratch_shapes=[pltpu.VMEM((1, D), _f32)],
)


def kernel(in_feat, edge_index, W1, b1, W2, b2, W3, b3):
    src_r = edge_index[0].reshape(NW, NCH, CH)
    dst_r = edge_index[1].reshape(NW, NCH, CH)

    dego_p, degi_p = _sc_degrees(src_r, dst_r)
    dego_a = dego_p[0].reshape(NPAD, 1)
    dego_b = dego_p[1].reshape(NPAD, 1)
    degi_a = degi_p[0].reshape(NPAD, 1)
    degi_b = degi_p[1].reshape(NPAD, 1)

    h1, ns, nd = _tc_norm_mm(dego_a, dego_b, degi_a, degi_b, in_feat, W1)

    agg1, c0 = _sc_agg_c0(h1, src_r, dst_r, nd.reshape(N))

    h2 = _tc_mid(agg1[0], agg1[1], nd, ns, b1.reshape(1, D), W2)

    agg2 = _sc_agg(h2, src_r, dst_r)

    out = _tc_final(agg2[0], agg2[1], nd, b2.reshape(1, D),
                    c0[0].reshape(NPAD, 1), c0[1].reshape(NPAD, 1), ns,
                    W3, b3.reshape(1, DO))
    return out


# speedup vs baseline: 7.9399x; 1.1865x over previous
import functools

import jax
import jax.numpy as jnp
from jax import lax
from jax.experimental import pallas as pl
from jax.experimental.pallas import tpu as pltpu
from jax.experimental.pallas import tpu_sc as plsc

N = 10000
E = 320000
D = 128
DO = 64

NC = 2
NS = 16
NW = NC * NS
EPT = E // NW
CH = 80
NCH = EPT // CH
NPAD = 10240
ZR = NPAD // NS
OR = N // NS

_mesh = plsc.VectorSubcoreMesh(core_axis_name="c", subcore_axis_name="s")

_f32 = jnp.float32
_i32 = jnp.int32


@functools.partial(
    pl.kernel,
    out_type=[jax.ShapeDtypeStruct((NC, NPAD), _f32),
              jax.ShapeDtypeStruct((NC, NPAD), _f32)],
    mesh=_mesh,
    compiler_params=pltpu.CompilerParams(needs_layout_passes=False),
    scratch_types=[
        pltpu.VMEM((NCH, CH), _i32),
        pltpu.VMEM((NCH, CH), _i32),
        pltpu.VMEM((NPAD,), _f32),
        pltpu.VMEM((NPAD,), _f32),
        pltpu.VMEM((ZR,), _f32),
        pltpu.VMEM((ZR,), _f32),
        pltpu.VMEM((ZR,), _f32),
        pltpu.VMEM((ZR,), _f32),
        pltpu.VMEM_SHARED((NS, 2, NPAD), _f32),
    ],
)
def _sc_degrees(src_hbm, dst_hbm, dego_out, degi_out,
                src_v, dst_v, dego_v, degi_v,
                acco_v, acci_v, tmpo_v, tmpi_v, slots):
    c = lax.axis_index("c")
    s = lax.axis_index("s")
    wid = s * NC + c
    pltpu.sync_copy(src_hbm.at[wid], src_v)
    pltpu.sync_copy(dst_hbm.at[wid], dst_v)
    z16 = jnp.zeros((16,), _f32)

    def zloop(i, _):
        dego_v[pl.ds(i * 16, 16)] = z16
        degi_v[pl.ds(i * 16, 16)] = z16
        return 0
    lax.fori_loop(0, NPAD // 16, zloop, 0)

    ones16 = jnp.ones((16,), _f32)

    def eloop(i, _):
        for j in range(CH // 16):
            sv = src_v[i, pl.ds(j * 16, 16)]
            dv = dst_v[i, pl.ds(j * 16, 16)]
            plsc.addupdate_scatter(dego_v, [sv], ones16)
            plsc.addupdate_scatter(degi_v, [dv], ones16)
        return 0
    lax.fori_loop(0, NCH, eloop, 0)

    pltpu.sync_copy(dego_v, slots.at[s, 0])
    pltpu.sync_copy(degi_v, slots.at[s, 1])
    plsc.subcore_barrier()

    pltpu.sync_copy(slots.at[0, 0, pl.ds(s * ZR, ZR)], acco_v)
    pltpu.sync_copy(slots.at[0, 1, pl.ds(s * ZR, ZR)], acci_v)

    def red(p, _):
        pltpu.sync_copy(slots.at[p, 0, pl.ds(s * ZR, ZR)], tmpo_v)
        pltpu.sync_copy(slots.at[p, 1, pl.ds(s * ZR, ZR)], tmpi_v)
        for k in range(ZR // 16):
            dk = pl.ds(k * 16, 16)
            acco_v[dk] = acco_v[dk] + tmpo_v[dk]
            acci_v[dk] = acci_v[dk] + tmpi_v[dk]
        return 0
    lax.fori_loop(1, NS, red, 0)

    pltpu.sync_copy(acco_v, dego_out.at[c, pl.ds(s * ZR, ZR)])
    pltpu.sync_copy(acci_v, degi_out.at[c, pl.ds(s * ZR, ZR)])


@functools.partial(
    pl.kernel,
    out_type=[jax.ShapeDtypeStruct((NC, NPAD, D), _f32),
              jax.ShapeDtypeStruct((NC, NPAD), _f32)],
    mesh=_mesh,
    compiler_params=pltpu.CompilerParams(needs_layout_passes=False),
    scratch_types=[
        pltpu.VMEM((CH,), _i32),
        pltpu.VMEM((CH,), _i32),
        pltpu.VMEM((CH, D), _f32),
        pltpu.VMEM((N,), _f32),
        pltpu.VMEM((NPAD,), _f32),
        pltpu.VMEM((ZR,), _f32),
        pltpu.VMEM((ZR,), _f32),
        pltpu.VMEM_SHARED((NPAD, D), _f32),
        pltpu.VMEM_SHARED((NS, NPAD), _f32),
        pltpu.SemaphoreType.DMA,
    ],
)
def _sc_agg_c0(h_hbm, src_hbm, dst_hbm, nd_hbm, agg_out, c0_out,
               src_c, dst_c, rows_v, nd_v, c_v,
               cacc_v, ctmp_v, sh_acc, cslots, gsem):
    c = lax.axis_index("c")
    s = lax.axis_index("s")
    wid = s * NC + c
    pltpu.sync_copy(nd_hbm, nd_v)

    z16 = jnp.zeros((16,), _f32)

    def zc(i, _):
        c_v[pl.ds(i * 16, 16)] = z16
        return 0
    lax.fori_loop(0, NPAD // 16, zc, 0)

    def zrow(i, _):
        for j in range(D // 16):
            rows_v[i, pl.ds(j * 16, 16)] = z16
        return 0
    lax.fori_loop(0, 16, zrow, 0)

    def zs(k, _):
        pltpu.sync_copy(rows_v.at[pl.ds(0, 16)],
                        sh_acc.at[pl.ds(s * ZR + k * 16, 16)])
        return 0
    lax.fori_loop(0, ZR // 16, zs, 0)
    plsc.subcore_barrier()

    def chunk(j, _):
        pltpu.sync_copy(src_hbm.at[wid, j], src_c)
        pltpu.sync_copy(dst_hbm.at[wid, j], dst_c)
        pltpu.async_copy(h_hbm.at[src_c], rows_v, gsem).wait()
        pltpu.sync_copy(rows_v, sh_acc.at[dst_c], add=True)
        for jj in range(CH // 16):
            sv = src_c[pl.ds(jj * 16, 16)]
            dv = dst_c[pl.ds(jj * 16, 16)]
            vals = plsc.load_gather(nd_v, [dv])
            plsc.addupdate_scatter(c_v, [sv], vals)
        return 0
    lax.fori_loop(0, NCH, chunk, 0)

    pltpu.sync_copy(c_v, cslots.at[s])
    plsc.subcore_barrier()

    pltpu.sync_copy(sh_acc.at[pl.ds(s * ZR, ZR)],
                    agg_out.at[c, pl.ds(s * ZR, ZR)])

    pltpu.sync_copy(cslots.at[0, pl.ds(s * ZR, ZR)], cacc_v)

    def cred(p, _):
        pltpu.sync_copy(cslots.at[p, pl.ds(s * ZR, ZR)], ctmp_v)
        for k in range(ZR // 16):
            dk = pl.ds(k * 16, 16)
            cacc_v[dk] = cacc_v[dk] + ctmp_v[dk]
        return 0
    lax.fori_loop(1, NS, cred, 0)

    pltpu.sync_copy(cacc_v, c0_out.at[c, pl.ds(s * ZR, ZR)])


@functools.partial(
    pl.kernel,
    out_type=jax.ShapeDtypeStruct((NC, NPAD, D), _f32),
    mesh=_mesh,
    compiler_params=pltpu.CompilerParams(needs_layout_passes=False),
    scratch_types=[
        pltpu.VMEM((CH,), _i32),
        pltpu.VMEM((CH,), _i32),
        pltpu.VMEM((CH,), _i32),
        pltpu.VMEM((CH,), _i32),
        pltpu.VMEM((CH, D), _f32),
        pltpu.VMEM((CH, D), _f32),
        pltpu.VMEM_SHARED((NPAD, D), _f32),
        pltpu.SemaphoreType.DMA,
    ],
)
def _sc_agg(h_hbm, src_hbm, dst_hbm, agg_out,
            srcA, dstA, srcB, dstB, rowsA, rowsB, sh_acc, gsem):
    c = lax.axis_index("c")
    s = lax.axis_index("s")
    wid = s * NC + c

    z16 = jnp.zeros((16,), _f32)

    def zrow(i, _):
        for j in range(D // 16):
            rowsA[i, pl.ds(j * 16, 16)] = z16
        return 0
    lax.fori_loop(0, 16, zrow, 0)

    def zs(k, _):
        pltpu.sync_copy(rowsA.at[pl.ds(0, 16)],
                        sh_acc.at[pl.ds(s * ZR + k * 16, 16)])
        return 0
    lax.fori_loop(0, ZR // 16, zs, 0)
    plsc.subcore_barrier()

    pltpu.sync_copy(src_hbm.at[wid, 0], srcA)
    pltpu.sync_copy(dst_hbm.at[wid, 0], dstA)
    pltpu.async_copy(h_hbm.at[srcA], rowsA, gsem)

    def pair(i, _):
        pltpu.sync_copy(src_hbm.at[wid, 2 * i + 1], srcB)
        pltpu.sync_copy(dst_hbm.at[wid, 2 * i + 1], dstB)
        pltpu.make_async_copy(h_hbm.at[srcA], rowsA, gsem).wait()
        pltpu.async_copy(h_hbm.at[srcB], rowsB, gsem)
        pltpu.sync_copy(rowsA, sh_acc.at[dstA], add=True)

        pltpu.sync_copy(src_hbm.at[wid, 2 * i + 2], srcA)
        pltpu.sync_copy(dst_hbm.at[wid, 2 * i + 2], dstA)
        pltpu.make_async_copy(h_hbm.at[srcB], rowsB, gsem).wait()
        pltpu.async_copy(h_hbm.at[srcA], rowsA, gsem)
        pltpu.sync_copy(rowsB, sh_acc.at[dstB], add=True)
        return 0
    lax.fori_loop(0, NCH // 2, pair, 0)

    pltpu.make_async_copy(h_hbm.at[srcA], rowsA, gsem).wait()
    pltpu.sync_copy(rowsA, sh_acc.at[dstA], add=True)
    plsc.subcore_barrier()

    pltpu.sync_copy(sh_acc.at[pl.ds(s * ZR, ZR)],
                    agg_out.at[c, pl.ds(s * ZR, ZR)])


_BR = 400
_GRID = N // _BR


def _tc_norm_mm_body(dego_a, dego_b, degi_a, degi_b, x, w, ho, nso, ndo):
    ns = lax.rsqrt(jnp.maximum(dego_a[...] + dego_b[...], 1.0))
    nd = lax.rsqrt(jnp.maximum(degi_a[...] + degi_b[...], 1.0))
    ho[...] = jnp.dot(x[...] * ns, w[...],
                      preferred_element_type=_f32,
                      precision=lax.Precision.HIGHEST)
    nso[...] = ns
    ndo[...] = nd


_col1 = pl.BlockSpec((_BR, 1), lambda i: (i, 0))
_rowb = pl.BlockSpec((_BR, D), lambda i: (i, 0))
_wfull = pl.BlockSpec((D, D), lambda i: (0, 0))

_tc_norm_mm = pl.pallas_call(
    _tc_norm_mm_body,
    grid=(_GRID,),
    in_specs=[_col1, _col1, _col1, _col1, _rowb, _wfull],
    out_specs=[_rowb, _col1, _col1],
    out_shape=[jax.ShapeDtypeStruct((N, D), _f32),
               jax.ShapeDtypeStruct((N, 1), _f32),
               jax.ShapeDtypeStruct((N, 1), _f32)],
)


def _tc_mid_body(agg_a, agg_b, nd, ns, b, w, ho):
    h = jnp.maximum((agg_a[...] + agg_b[...]) * nd[...] + b[...], 0.0)
    ho[...] = jnp.dot(h * ns[...], w[...],
                      preferred_element_type=_f32,
                      precision=lax.Precision.HIGHEST)


_bfull = pl.BlockSpec((1, D), lambda i: (0, 0))

_tc_mid = pl.pallas_call(
    _tc_mid_body,
    grid=(_GRID,),
    in_specs=[_rowb, _rowb, _col1, _col1, _bfull, _wfull],
    out_specs=_rowb,
    out_shape=jax.ShapeDtypeStruct((N, D), _f32),
)


def _tc_final_body(agg_a, agg_b, nd, b2, c0a, c0b, ns, w3, b3, out, racc):
    i = pl.program_id(0)

    @pl.when(i == 0)
    def _():
        racc[...] = jnp.zeros_like(racc)

    h2 = jnp.maximum((agg_a[...] + agg_b[...]) * nd[...] + b2[...], 0.0)
    wvec = (c0a[...] + c0b[...]) * ns[...]
    racc[...] += jnp.sum(h2 * wvec, axis=0, keepdims=True)

    @pl.when(i == pl.num_programs(0) - 1)
    def _():
        out[...] = jnp.dot(racc[...] * (1.0 / N), w3[...],
                           preferred_element_type=_f32,
                           precision=lax.Precision.HIGHEST) + b3[...]


_tc_final = pl.pallas_call(
    _tc_final_body,
    grid=(_GRID,),
    in_specs=[_rowb, _rowb, _col1, _bfull, _col1, _col1, _col1,
              pl.BlockSpec((D, DO), lambda i: (0, 0)),
              pl.BlockSpec((1, DO), lambda i: (0, 0))],
    out_specs=pl.BlockSpec((1, DO), lambda i: (0, 0)),
    out_shape=jax.ShapeDtypeStruct((1, DO), _f32),
    scratch_shapes=[pltpu.VMEM((1, D), _f32)],
)


def kernel(in_feat, edge_index, W1, b1, W2, b2, W3, b3):
    src_r = edge_index[0].reshape(NW, NCH, CH)
    dst_r = edge_index[1].reshape(NW, NCH, CH)

    dego_p, degi_p = _sc_degrees(src_r, dst_r)
    dego_a = dego_p[0].reshape(NPAD, 1)
    dego_b = dego_p[1].reshape(NPAD, 1)
    degi_a = degi_p[0].reshape(NPAD, 1)
    degi_b = degi_p[1].reshape(NPAD, 1)

    h1, ns, nd = _tc_norm_mm(dego_a, dego_b, degi_a, degi_b, in_feat, W1)

    agg1 = _sc_agg(h1, src_r, dst_r)

    h2 = _tc_mid(agg1[0], agg1[1], nd, ns, b1.reshape(1, D), W2)

    agg2, c0 = _sc_agg_c0(h2, src_r, dst_r, nd.reshape(N))

    out = _tc_final(agg2[0], agg2[1], nd, b2.reshape(1, D),
                    c0[0].reshape(NPAD, 1), c0[1].reshape(NPAD, 1), ns,
                    W3, b3.reshape(1, DO))
    return out


# speedup vs baseline: 11.8307x; 1.4900x over previous
import functools

import jax
import jax.numpy as jnp
from jax import lax
from jax.experimental import pallas as pl
from jax.experimental.pallas import tpu as pltpu
from jax.experimental.pallas import tpu_sc as plsc

N = 10000
E = 320000
D = 128
DO = 64

NC = 2
NS = 16
NW = NC * NS
EPT = E // NW
CH = 80
NCH = EPT // CH
NPAD = 10240
CHP = 128
EPTP = 10240
NCHP = EPTP // CHP
ZR = NPAD // NS
OR = N // NS

_mesh = plsc.VectorSubcoreMesh(core_axis_name="c", subcore_axis_name="s")

_f32 = jnp.float32
_i32 = jnp.int32


@functools.partial(
    pl.kernel,
    out_type=[jax.ShapeDtypeStruct((NC, NPAD), _f32),
              jax.ShapeDtypeStruct((NC, NPAD), _f32)],
    mesh=_mesh,
    compiler_params=pltpu.CompilerParams(needs_layout_passes=False),
    scratch_types=[
        pltpu.VMEM((NCH, CH), _i32),
        pltpu.VMEM((NCH, CH), _i32),
        pltpu.VMEM((NPAD,), _f32),
        pltpu.VMEM((NPAD,), _f32),
        pltpu.VMEM((ZR,), _f32),
        pltpu.VMEM((ZR,), _f32),
        pltpu.VMEM((ZR,), _f32),
        pltpu.VMEM((ZR,), _f32),
        pltpu.VMEM_SHARED((NS, 2, NPAD), _f32),
    ],
)
def _sc_degrees(src_hbm, dst_hbm, dego_out, degi_out,
                src_v, dst_v, dego_v, degi_v,
                acco_v, acci_v, tmpo_v, tmpi_v, slots):
    c = lax.axis_index("c")
    s = lax.axis_index("s")
    wid = s * NC + c
    pltpu.sync_copy(src_hbm.at[wid], src_v)
    pltpu.sync_copy(dst_hbm.at[wid], dst_v)
    z16 = jnp.zeros((16,), _f32)

    def zloop(i, _):
        dego_v[pl.ds(i * 16, 16)] = z16
        degi_v[pl.ds(i * 16, 16)] = z16
        return 0
    lax.fori_loop(0, NPAD // 16, zloop, 0)

    ones16 = jnp.ones((16,), _f32)

    def eloop(i, _):
        for j in range(CH // 16):
            sv = src_v[i, pl.ds(j * 16, 16)]
            dv = dst_v[i, pl.ds(j * 16, 16)]
            plsc.addupdate_scatter(dego_v, [sv], ones16)
            plsc.addupdate_scatter(degi_v, [dv], ones16)
        return 0
    lax.fori_loop(0, NCH, eloop, 0)

    pltpu.sync_copy(dego_v, slots.at[s, 0])
    pltpu.sync_copy(degi_v, slots.at[s, 1])
    plsc.subcore_barrier()

    pltpu.sync_copy(slots.at[0, 0, pl.ds(s * ZR, ZR)], acco_v)
    pltpu.sync_copy(slots.at[0, 1, pl.ds(s * ZR, ZR)], acci_v)

    def red(p, _):
        pltpu.sync_copy(slots.at[p, 0, pl.ds(s * ZR, ZR)], tmpo_v)
        pltpu.sync_copy(slots.at[p, 1, pl.ds(s * ZR, ZR)], tmpi_v)
        for k in range(ZR // 16):
            dk = pl.ds(k * 16, 16)
            acco_v[dk] = acco_v[dk] + tmpo_v[dk]
            acci_v[dk] = acci_v[dk] + tmpi_v[dk]
        return 0
    lax.fori_loop(1, NS, red, 0)

    pltpu.sync_copy(acco_v, dego_out.at[c, pl.ds(s * ZR, ZR)])
    pltpu.sync_copy(acci_v, degi_out.at[c, pl.ds(s * ZR, ZR)])


@functools.partial(
    pl.kernel,
    out_type=[jax.ShapeDtypeStruct((NC, NPAD, D), _f32),
              jax.ShapeDtypeStruct((NW, NPAD), _f32)],
    mesh=_mesh,
    compiler_params=pltpu.CompilerParams(needs_layout_passes=False),
    scratch_types=[
        pltpu.VMEM((CHP,), _i32),
        pltpu.VMEM((CHP,), _i32),
        pltpu.VMEM((CHP,), _i32),
        pltpu.VMEM((CHP,), _i32),
        pltpu.VMEM((CHP, D), _f32),
        pltpu.VMEM((CHP, D), _f32),
        pltpu.VMEM((CHP,), _f32),
        pltpu.VMEM((CHP,), _f32),
        pltpu.VMEM((NPAD,), _f32),
        pltpu.VMEM_SHARED((NPAD, D), _f32),
        pltpu.SemaphoreType.DMA,
        pltpu.SemaphoreType.DMA,
        pltpu.SemaphoreType.DMA,
        pltpu.SemaphoreType.DMA,
        pltpu.SemaphoreType.DMA,
    ],
)
def _sc_agg_c0(h_hbm, src_hbm, dst_hbm, nd_hbm, agg_out, c0_out,
               srcA, dstA, srcB, dstB, rowsA, rowsB, valsA, valsB, c_v,
               sh_acc, gsem, vsemA, vsemB, ssemA, ssemB):
    c = lax.axis_index("c")
    s = lax.axis_index("s")
    wid = s * NC + c

    z16 = jnp.zeros((16,), _f32)

    def zc(i, _):
        c_v[pl.ds(i * 16, 16)] = z16
        return 0
    lax.fori_loop(0, NPAD // 16, zc, 0)

    def zrow(i, _):
        for j in range(D // 16):
            rowsA[i, pl.ds(j * 16, 16)] = z16
        return 0
    lax.fori_loop(0, 16, zrow, 0)

    def zs(k, _):
        pltpu.sync_copy(rowsA.at[pl.ds(0, 16)],
                        sh_acc.at[pl.ds(s * ZR + k * 16, 16)])
        return 0
    lax.fori_loop(0, ZR // 16, zs, 0)
    plsc.subcore_barrier()

    def cpass(src_c, vals_v):
        for jj in range(CHP // 16):
            sv = src_c[pl.ds(jj * 16, 16)]
            vv = vals_v[pl.ds(jj * 16, 16)]
            plsc.addupdate_scatter(c_v, [sv], vv)

    pltpu.sync_copy(src_hbm.at[wid, 0], srcA)
    pltpu.sync_copy(dst_hbm.at[wid, 0], dstA)
    pltpu.async_copy(nd_hbm.at[dstA], valsA, vsemA)
    pltpu.async_copy(h_hbm.at[srcA], rowsA, gsem)

    def pair(i, _):
        @pl.when(i > 0)
        def _():
            pltpu.make_async_copy(rowsB, sh_acc.at[dstB], ssemB).wait()
        pltpu.sync_copy(src_hbm.at[wid, 2 * i + 1], srcB)
        pltpu.sync_copy(dst_hbm.at[wid, 2 * i + 1], dstB)
        pltpu.async_copy(nd_hbm.at[dstB], valsB, vsemB)
        pltpu.make_async_copy(h_hbm.at[srcA], rowsA, gsem).wait()
        pltpu.async_copy(h_hbm.at[srcB], rowsB, gsem)
        pltpu.async_copy(rowsA, sh_acc.at[dstA], ssemA, add=True)
        pltpu.make_async_copy(nd_hbm.at[dstA], valsA, vsemA).wait()
        cpass(srcA, valsA)

        pltpu.make_async_copy(rowsA, sh_acc.at[dstA], ssemA).wait()

        @pl.when(i < NCHP // 2 - 1)
        def _():
            pltpu.sync_copy(src_hbm.at[wid, 2 * i + 2], srcA)
            pltpu.sync_copy(dst_hbm.at[wid, 2 * i + 2], dstA)
            pltpu.async_copy(nd_hbm.at[dstA], valsA, vsemA)
        pltpu.make_async_copy(h_hbm.at[srcB], rowsB, gsem).wait()

        @pl.when(i < NCHP // 2 - 1)
        def _():
            pltpu.async_copy(h_hbm.at[srcA], rowsA, gsem)
        pltpu.async_copy(rowsB, sh_acc.at[dstB], ssemB, add=True)
        pltpu.make_async_copy(nd_hbm.at[dstB], valsB, vsemB).wait()
        cpass(srcB, valsB)
        return 0
    lax.fori_loop(0, NCHP // 2, pair, 0)

    pltpu.make_async_copy(rowsB, sh_acc.at[dstB], ssemB).wait()
    plsc.subcore_barrier()

    pltpu.sync_copy(sh_acc.at[pl.ds(s * ZR, ZR)],
                    agg_out.at[c, pl.ds(s * ZR, ZR)])
    pltpu.sync_copy(c_v, c0_out.at[wid])


@functools.partial(
    pl.kernel,
    out_type=jax.ShapeDtypeStruct((NC, NPAD, D), _f32),
    mesh=_mesh,
    compiler_params=pltpu.CompilerParams(needs_layout_passes=False),
    scratch_types=[
        pltpu.VMEM((EPTP,), _i32),
        pltpu.VMEM((CHP,), _i32),
        pltpu.VMEM((CHP,), _i32),
        pltpu.VMEM((2, CHP, D), _f32),
        pltpu.VMEM_SHARED((NPAD, D), _f32),
        pltpu.SemaphoreType.DMA,
        pltpu.SemaphoreType.DMA,
    ],
)
def _sc_agg(h_hbm, srcf_hbm, dst_hbm, agg_out,
            src_v, dstA, dstB, rows2, sh_acc, gsem, ssem):
    c = lax.axis_index("c")
    s = lax.axis_index("s")
    wid = s * NC + c
    pltpu.sync_copy(srcf_hbm.at[wid], src_v)

    z16 = jnp.zeros((16,), _f32)

    def zrow(i, _):
        for j in range(D // 16):
            rows2[0, i, pl.ds(j * 16, 16)] = z16
        return 0
    lax.fori_loop(0, 16, zrow, 0)

    def zs(k, _):
        pltpu.sync_copy(rows2.at[0, pl.ds(0, 16)],
                        sh_acc.at[pl.ds(s * ZR + k * 16, 16)])
        return 0
    lax.fori_loop(0, ZR // 16, zs, 0)
    plsc.subcore_barrier()

    pltpu.async_copy(h_hbm.at[src_v.at[pl.ds(0, CHP)]], rows2.at[0], gsem)

    def chunk(j, _):
        cur = lax.rem(j, 2)
        pltpu.make_async_copy(h_hbm.at[src_v.at[pl.ds(j * CHP, CHP)]],
                              rows2.at[cur], gsem).wait()

        @pl.when(j > 0)
        def _():
            prv = 1 - cur

            @pl.when(lax.rem(j, 2) == 0)
            def _():
                pltpu.make_async_copy(rows2.at[prv], sh_acc.at[dstB],
                                      ssem).wait()

            @pl.when(lax.rem(j, 2) == 1)
            def _():
                pltpu.make_async_copy(rows2.at[prv], sh_acc.at[dstA],
                                      ssem).wait()

        @pl.when(j < NCHP - 1)
        def _():
            pltpu.async_copy(
                h_hbm.at[src_v.at[pl.ds((j + 1) * CHP, CHP)]],
                rows2.at[1 - cur], gsem)

        @pl.when(lax.rem(j, 2) == 0)
        def _():
            pltpu.sync_copy(dst_hbm.at[wid, j], dstA)
            pltpu.async_copy(rows2.at[cur], sh_acc.at[dstA], ssem, add=True)

        @pl.when(lax.rem(j, 2) == 1)
        def _():
            pltpu.sync_copy(dst_hbm.at[wid, j], dstB)
            pltpu.async_copy(rows2.at[cur], sh_acc.at[dstB], ssem, add=True)
        return 0
    lax.fori_loop(0, NCHP, chunk, 0)
    pltpu.make_async_copy(rows2.at[(NCHP - 1) % 2],
                          sh_acc.at[dstB], ssem).wait()
    plsc.subcore_barrier()

    pltpu.sync_copy(sh_acc.at[pl.ds(s * ZR, ZR)],
                    agg_out.at[c, pl.ds(s * ZR, ZR)])


_BR = 400
_GRID = N // _BR


def _tc_norm_mm_body(dego_a, dego_b, degi_a, degi_b, x, w, ho, nso, ndo):
    ns = lax.rsqrt(jnp.maximum(dego_a[...] + dego_b[...], 1.0))
    nd = lax.rsqrt(jnp.maximum(degi_a[...] + degi_b[...], 1.0))
    ho[...] = jnp.dot(x[...] * ns, w[...],
                      preferred_element_type=_f32,
                      precision=lax.Precision.HIGHEST)
    nso[...] = ns
    ndo[...] = nd


_col1 = pl.BlockSpec((_BR, 1), lambda i: (i, 0))
_rowb = pl.BlockSpec((_BR, D), lambda i: (i, 0))
_wfull = pl.BlockSpec((D, D), lambda i: (0, 0))

_tc_norm_mm = pl.pallas_call(
    _tc_norm_mm_body,
    grid=(_GRID,),
    in_specs=[_col1, _col1, _col1, _col1, _rowb, _wfull],
    out_specs=[_rowb, _col1, _col1],
    out_shape=[jax.ShapeDtypeStruct((NPAD, D), _f32),
               jax.ShapeDtypeStruct((NPAD, 1), _f32),
               jax.ShapeDtypeStruct((NPAD, 1), _f32)],
)


def _tc_mid_body(agg_a, agg_b, nd, ns, b, w, c0p, ho, c0so):
    h = jnp.maximum((agg_a[...] + agg_b[...]) * nd[...] + b[...], 0.0)
    ho[...] = jnp.dot(h * ns[...], w[...],
                      preferred_element_type=_f32,
                      precision=lax.Precision.HIGHEST)
    c0so[...] = jnp.sum(c0p[...], axis=0, keepdims=True)


_bfull = pl.BlockSpec((1, D), lambda i: (0, 0))

_tc_mid = pl.pallas_call(
    _tc_mid_body,
    grid=(_GRID,),
    in_specs=[_rowb, _rowb, _col1, _col1, _bfull, _wfull,
              pl.BlockSpec((NW, NPAD // 8), lambda i: (0, i % 8))],
    out_specs=[_rowb, pl.BlockSpec((1, NPAD // 8), lambda i: (0, i % 8))],
    out_shape=[jax.ShapeDtypeStruct((NPAD, D), _f32),
               jax.ShapeDtypeStruct((1, NPAD), _f32)],
)


def _tc_final_body(agg_a, agg_b, nd, b2, c0s, ns, w3, b3, out, racc):
    i = pl.program_id(0)

    @pl.when(i == 0)
    def _():
        racc[...] = jnp.zeros_like(racc)

    h2 = jnp.maximum((agg_a[...] + agg_b[...]) * nd[...] + b2[...], 0.0)
    wvec = c0s[...] * ns[...]
    racc[...] += jnp.sum(h2 * wvec, axis=0, keepdims=True)

    @pl.when(i == pl.num_programs(0) - 1)
    def _():
        out[...] = jnp.dot(racc[...] * (1.0 / N), w3[...],
                           preferred_element_type=_f32,
                           precision=lax.Precision.HIGHEST) + b3[...]


_tc_final = pl.pallas_call(
    _tc_final_body,
    grid=(_GRID,),
    in_specs=[_rowb, _rowb, _col1, _bfull, _col1, _col1,
              pl.BlockSpec((D, DO), lambda i: (0, 0)),
              pl.BlockSpec((1, DO), lambda i: (0, 0))],
    out_specs=pl.BlockSpec((1, DO), lambda i: (0, 0)),
    out_shape=jax.ShapeDtypeStruct((1, DO), _f32),
    scratch_shapes=[pltpu.VMEM((1, D), _f32)],
)


def kernel(in_feat, edge_index, W1, b1, W2, b2, W3, b3):
    src_r = edge_index[0].reshape(NW, NCH, CH)
    dst_r = edge_index[1].reshape(NW, NCH, CH)

    dego_p, degi_p = _sc_degrees(src_r, dst_r)
    dego_a = dego_p[0].reshape(NPAD, 1)
    dego_b = dego_p[1].reshape(NPAD, 1)
    degi_a = degi_p[0].reshape(NPAD, 1)
    degi_b = degi_p[1].reshape(NPAD, 1)

    h1, ns, nd = _tc_norm_mm(dego_a, dego_b, degi_a, degi_b, in_feat, W1)

    pad_idx = jnp.broadcast_to(
        N + jnp.arange(EPTP - EPT, dtype=jnp.int32) % (NPAD - N),
        (NW, EPTP - EPT))
    srcp = jnp.concatenate(
        [edge_index[0].reshape(NW, EPT), pad_idx], axis=1
    ).reshape(NW, NCHP, CHP)
    dstp = jnp.concatenate(
        [edge_index[1].reshape(NW, EPT), pad_idx], axis=1
    ).reshape(NW, NCHP, CHP)

    agg1, c0p = _sc_agg_c0(h1, srcp, dstp, nd.reshape(NPAD))

    h2, c0s = _tc_mid(agg1[0], agg1[1], nd, ns, b1.reshape(1, D), W2, c0p)

    agg2 = _sc_agg(h2, srcp.reshape(NW, EPTP), dstp)

    out = _tc_final(agg2[0], agg2[1], nd, b2.reshape(1, D),
                    c0s.reshape(NPAD, 1), ns,
                    W3, b3.reshape(1, DO))
    return out
